# baseline (speedup 1.0000x reference)
import functools

import jax
import jax.numpy as jnp
from jax import lax
from jax.experimental import pallas as pl
from jax.experimental.pallas import tpu as pltpu
from jax.experimental.pallas import tpu_sc as plsc

N = 10000
E = 160000
D = 256

NW = 32
NPS = 320
NPAD = NW * NPS
ACC_ROWS = 328
SENT = 320
CHUNK = 2000
FLUSH = 2016
EB = 160768

_mesh = plsc.VectorSubcoreMesh(core_axis_name="c", subcore_axis_name="s")


def _wid():
    return lax.axis_index("s") * 2 + lax.axis_index("c")


def _bucket_body(src_ref, dst_ref, bucket_ref, counts_ref,
                 srcbuf, dstbuf, buf, cntb):
    w = _wid()
    base = w * NPS
    sent_vec = jnp.full((16,), SENT, dtype=jnp.int32)

    def chunk_body(c, total):
        pltpu.sync_copy(src_ref.at[pl.ds(c * CHUNK, CHUNK)], srcbuf)
        pltpu.sync_copy(dst_ref.at[pl.ds(c * CHUNK, CHUNK)], dstbuf)

        def scan_body(i, cnt):
            s16 = srcbuf[pl.ds(i * 16, 16)]
            d16 = dstbuf[pl.ds(i * 16, 16)]
            msk = (d16 >= base) & (d16 < base + NPS)
            pk = (s16 << 9) | (d16 - base)
            mi = jnp.where(msk, 1, 0).astype(jnp.int32)
            pos = plsc.cumsum(mi) - 1 + cnt
            plsc.store_scatter(buf, [pos], pk, mask=msk)
            npop = plsc.all_reduce_population_count(msk)
            return cnt + jnp.max(npop)

        cnt_c = lax.fori_loop(0, CHUNK // 16, scan_body, 0)
        buf[pl.ds(cnt_c, 16)] = sent_vec
        cnt8 = (cnt_c + 7) & (-8)
        off = pl.multiple_of(w * EB + total, 8)
        pltpu.sync_copy(buf, bucket_ref.at[pl.ds(off, FLUSH)])
        return total + cnt8

    total = lax.fori_loop(0, E // CHUNK, chunk_body, 0)
    for k in range(8):
        buf[pl.ds(k * 16, 16)] = sent_vec
    off = pl.multiple_of(w * EB + total, 8)
    pltpu.sync_copy(buf.at[pl.ds(0, 128)], bucket_ref.at[pl.ds(off, 128)])
    count_out = (total + 127) & (-128)
    cntb[...] = jnp.full((16,), count_out, dtype=jnp.int32)
    pltpu.sync_copy(cntb, counts_ref.at[pl.ds(pl.multiple_of(w * 16, 8), 16)])


_bucket = pl.kernel(
    _bucket_body,
    compiler_params=pltpu.CompilerParams(needs_layout_passes=False),
    out_type=(
        jax.ShapeDtypeStruct((NW * EB,), jnp.int32),
        jax.ShapeDtypeStruct((NW * 16,), jnp.int32),
    ),
    mesh=_mesh,
    scratch_types=[
        pltpu.VMEM((CHUNK,), jnp.int32),
        pltpu.VMEM((CHUNK,), jnp.int32),
        pltpu.VMEM((FLUSH,), jnp.int32),
        pltpu.VMEM((16,), jnp.int32),
    ],
)


GB = 128


def _segmax_body(p_ref, bucket_ref, counts_ref, out_ref,
                 pkbuf, srcv, dlv, rows, acc, cntb, sem0, sem1):
    w = _wid()
    zeros16 = jnp.zeros((16,), dtype=jnp.int32)

    def zero_body(r, _):
        for k in range(8):
            acc[r, pl.ds(k * 16, 16)] = zeros16
        return 0

    lax.fori_loop(0, ACC_ROWS, zero_body, 0)

    pltpu.sync_copy(counts_ref.at[pl.ds(pl.multiple_of(w * 16, 8), 16)], cntb)
    count = cntb[...][0]
    nb = count >> 7

    def load_pk_chunk(cix):
        off = pl.multiple_of(w * EB + cix * (16 * GB), 8)
        pltpu.sync_copy(bucket_ref.at[pl.ds(off, 16 * GB)], pkbuf)

    def prep(b, slot):
        base = (b & 15) * GB
        for k in range(GB // 16):
            pk = pkbuf[pl.ds(base + k * 16, 16)]
            srcv[slot, pl.ds(k * 16, 16)] = lax.shift_right_logical(pk, 9)
            dlv[slot, pl.ds(k * 16, 16)] = pk & 511

    def fire(b, slot):
        @pl.when(slot == 0)
        def _():
            prep(b, 0)
            pltpu.async_copy(p_ref.at[srcv.at[0]], rows.at[0], sem0)

        @pl.when(slot != 0)
        def _():
            prep(b, 1)
            pltpu.async_copy(p_ref.at[srcv.at[1]], rows.at[1], sem1)

    def wait(slot):
        @pl.when(slot == 0)
        def _():
            pltpu.make_async_copy(p_ref.at[srcv.at[0]], rows.at[0],
                                  sem0).wait()

        @pl.when(slot != 0)
        def _():
            pltpu.make_async_copy(p_ref.at[srcv.at[1]], rows.at[1],
                                  sem1).wait()

    @pl.when(nb >= 1)
    def _():
        load_pk_chunk(0)
        fire(0, 0)

    def block_body(b, _):
        slot = b & 1

        @pl.when(b + 1 < nb)
        def _():
            @pl.when(((b + 1) & 15) == 0)
            def _():
                load_pk_chunk((b + 1) >> 4)

            fire(b + 1, slot ^ 1)

        wait(slot)

        def accum(rows_s, dlv_s):
            def qbody(q, _):
                dl16 = dlv_s[pl.ds(q * 16, 16)]
                dl = dl16[0]
                for j in range(16):
                    qj = q * 16 + j
                    dl_next = dl16[j + 1] if j < 15 else dl
                    av = [acc[dl, pl.ds(0, 16)], acc[dl, pl.ds(16, 16)]]
                    rv = [rows_s[qj, pl.ds(0, 16)],
                          rows_s[qj, pl.ds(16, 16)]]
                    for k in range(8):
                        if k + 2 < 8:
                            av.append(acc[dl, pl.ds((k + 2) * 16, 16)])
                            rv.append(rows_s[qj, pl.ds((k + 2) * 16, 16)])
                        m = jnp.maximum(plsc.bitcast(av[k], jnp.bfloat16),
                                        plsc.bitcast(rv[k], jnp.bfloat16))
                        acc[dl, pl.ds(k * 16, 16)] = plsc.bitcast(
                            m, jnp.int32)
                    dl = dl_next
                return 0

            lax.fori_loop(0, GB // 16, qbody, 0)

        @pl.when(slot == 0)
        def _():
            accum(rows.at[0], dlv.at[0])

        @pl.when(slot != 0)
        def _():
            accum(rows.at[1], dlv.at[1])

        return 0

    lax.fori_loop(0, nb, block_body, 0)
    pltpu.sync_copy(acc.at[pl.ds(0, NPS)],
                    out_ref.at[pl.ds(pl.multiple_of(w * NPS, 8), NPS)])


_segmax = pl.kernel(
    _segmax_body,
    compiler_params=pltpu.CompilerParams(needs_layout_passes=False),
    out_type=jax.ShapeDtypeStruct((NPAD, D // 2), jnp.int32),
    mesh=_mesh,
    scratch_types=[
        pltpu.VMEM((16 * GB,), jnp.int32),
        pltpu.VMEM((2, GB), jnp.int32),
        pltpu.VMEM((2, GB), jnp.int32),
        pltpu.VMEM((2, GB, D // 2), jnp.int32),
        pltpu.VMEM((ACC_ROWS, D // 2), jnp.int32),
        pltpu.VMEM((16,), jnp.int32),
        pltpu.SemaphoreType.DMA,
        pltpu.SemaphoreType.DMA,
    ],
)


_RB = 2000


def _pool_body(x_ref, wp_ref, bp_ref, ws_ref, b_ref, p_ref, s_ref):
    x = x_ref[...]
    p = jnp.maximum(
        jnp.dot(x, wp_ref[...], preferred_element_type=jnp.float32)
        + bp_ref[...], 0.0)
    p_ref[...] = p.astype(jnp.bfloat16)
    s_ref[...] = (jnp.dot(x, ws_ref[...], preferred_element_type=jnp.float32)
                  + b_ref[...])


def _pool(x, wp, bp, ws, b):
    return pl.pallas_call(
        _pool_body,
        grid=(N // _RB,),
        in_specs=[
            pl.BlockSpec((_RB, D), lambda i: (i, 0)),
            pl.BlockSpec((D, D), lambda i: (0, 0)),
            pl.BlockSpec((1, D), lambda i: (0, 0)),
            pl.BlockSpec((D, D), lambda i: (0, 0)),
            pl.BlockSpec((1, D), lambda i: (0, 0)),
        ],
        out_specs=[
            pl.BlockSpec((_RB, D), lambda i: (i, 0)),
            pl.BlockSpec((_RB, D), lambda i: (i, 0)),
        ],
        out_shape=[
            jax.ShapeDtypeStruct((N, D), jnp.bfloat16),
            jax.ShapeDtypeStruct((N, D), jnp.float32),
        ],
    )(x, wp, bp.reshape(1, D), ws, b.reshape(1, D))


def _comb_body(s_ref, g_ref, wn_ref, o_ref, *, act):
    o = s_ref[...] + jnp.dot(g_ref[...], wn_ref[...],
                             preferred_element_type=jnp.float32)
    o_ref[...] = jnp.tanh(o) if act else o


def _comb(s, g, wn, act):
    return pl.pallas_call(
        functools.partial(_comb_body, act=act),
        grid=(N // _RB,),
        in_specs=[
            pl.BlockSpec((_RB, D), lambda i: (i, 0)),
            pl.BlockSpec((_RB, D), lambda i: (i, 0)),
            pl.BlockSpec((D, D), lambda i: (0, 0)),
        ],
        out_specs=pl.BlockSpec((_RB, D), lambda i: (i, 0)),
        out_shape=jax.ShapeDtypeStruct((N, D), jnp.float32),
    )(s, g, wn)


def kernel(x, edge_index,
           W_pool1, b_pool1, W_self1, W_neigh1, b1,
           W_pool2, b_pool2, W_self2, W_neigh2, b2,
           W_pool3, b_pool3, W_self3, W_neigh3, b3):
    bucket, counts = _bucket(edge_index[0], edge_index[1])

    def layer(h, wp, bp, ws, wn, b, act):
        p, s = _pool(h, wp, bp, ws, b)
        p32 = lax.bitcast_convert_type(p.reshape(N, D // 2, 2), jnp.int32)
        g32 = _segmax(p32, bucket, counts)
        g = lax.bitcast_convert_type(g32, jnp.bfloat16).reshape(NPAD, D)[:N]
        return _comb(s, g, wn, act)

    h = layer(x, W_pool1, b_pool1, W_self1, W_neigh1, b1, True)
    h = layer(h, W_pool2, b_pool2, W_self2, W_neigh2, b2, True)
    h = layer(h, W_pool3, b_pool3, W_self3, W_neigh3, b3, False)
    return h


# speedup vs baseline: 2.6681x; 2.6681x over previous
import functools

import jax
import jax.numpy as jnp
from jax import lax
from jax.experimental import pallas as pl
from jax.experimental.pallas import tpu as pltpu
from jax.experimental.pallas import tpu_sc as plsc

N = 10000
E = 160000
D = 256

NW = 32
NPS = 320
NPAD = NW * NPS
HW = 64
ACC_ROWS = 168
SENT = 320
CHUNK = 2000
FLUSH = 2016
EB = 160768

_mesh = plsc.VectorSubcoreMesh(core_axis_name="c", subcore_axis_name="s")


def _wid():
    return lax.axis_index("s") * 2 + lax.axis_index("c")


def _bucket_body(src_ref, dst_ref, bucket_ref, counts_ref,
                 srcbuf, dstbuf, buf, cntb):
    w = _wid()
    base = w * NPS
    sent_vec = jnp.full((16,), SENT, dtype=jnp.int32)

    def chunk_body(c, total):
        pltpu.sync_copy(src_ref.at[pl.ds(c * CHUNK, CHUNK)], srcbuf)
        pltpu.sync_copy(dst_ref.at[pl.ds(c * CHUNK, CHUNK)], dstbuf)

        def scan_body(i, cnt):
            s16 = srcbuf[pl.ds(i * 16, 16)]
            d16 = dstbuf[pl.ds(i * 16, 16)]
            msk = (d16 >= base) & (d16 < base + NPS)
            pk = (s16 << 9) | (d16 - base)
            mi = jnp.where(msk, 1, 0).astype(jnp.int32)
            pos = plsc.cumsum(mi) - 1 + cnt
            plsc.store_scatter(buf, [pos], pk, mask=msk)
            npop = plsc.all_reduce_population_count(msk)
            return cnt + jnp.max(npop)

        cnt_c = lax.fori_loop(0, CHUNK // 16, scan_body, 0)
        buf[pl.ds(cnt_c, 16)] = sent_vec
        cnt8 = (cnt_c + 7) & (-8)
        off = pl.multiple_of(w * EB + total, 8)
        pltpu.sync_copy(buf, bucket_ref.at[pl.ds(off, FLUSH)])
        return total + cnt8

    total = lax.fori_loop(0, E // CHUNK, chunk_body, 0)
    for k in range(8):
        buf[pl.ds(k * 16, 16)] = sent_vec
    off = pl.multiple_of(w * EB + total, 8)
    pltpu.sync_copy(buf.at[pl.ds(0, 128)], bucket_ref.at[pl.ds(off, 128)])
    count_out = (total + 127) & (-128)
    cntb[...] = jnp.full((16,), count_out, dtype=jnp.int32)
    pltpu.sync_copy(cntb, counts_ref.at[pl.ds(pl.multiple_of(w * 16, 8), 16)])


_bucket = pl.kernel(
    _bucket_body,
    compiler_params=pltpu.CompilerParams(needs_layout_passes=False),
    out_type=(
        jax.ShapeDtypeStruct((NW * EB,), jnp.int32),
        jax.ShapeDtypeStruct((NW * 16,), jnp.int32),
    ),
    mesh=_mesh,
    scratch_types=[
        pltpu.VMEM((CHUNK,), jnp.int32),
        pltpu.VMEM((CHUNK,), jnp.int32),
        pltpu.VMEM((FLUSH,), jnp.int32),
        pltpu.VMEM((16,), jnp.int32),
    ],
)


GB = 128


def _segmax_body(p_ref, bucket_ref, counts_ref, out_ref,
                 pkbuf, srcv, dlv, rows, acc, cntb, shared, sem0, sem1):
    w = _wid()
    sid = lax.axis_index("s")
    zeros16 = jnp.zeros((16,), dtype=jnp.int32)

    soff = pl.multiple_of(sid * (NPAD // 32), 8)
    pltpu.sync_copy(p_ref.at[pl.ds(soff, NPAD // 32)],
                    shared.at[pl.ds(soff, NPAD // 32)])

    def zero_body(r, _):
        for k in range(8):
            acc[r, pl.ds(k * 16, 16)] = zeros16
        return 0

    lax.fori_loop(0, ACC_ROWS, zero_body, 0)
    plsc.subcore_barrier()

    pltpu.sync_copy(counts_ref.at[pl.ds(pl.multiple_of(w * 16, 8), 16)], cntb)
    count = cntb[...][0]
    nb = count >> 7

    def load_pk_chunk(cix):
        off = pl.multiple_of(w * EB + cix * (16 * GB), 8)
        pltpu.sync_copy(bucket_ref.at[pl.ds(off, 16 * GB)], pkbuf)

    def prep(bb, slot):
        base = (bb & 15) * GB
        for k in range(GB // 16):
            pk = pkbuf[pl.ds(base + k * 16, 16)]
            s16 = lax.shift_right_logical(pk, 9)
            dl16 = pk & 511
            srcv[slot, pl.ds(k * 16, 16)] = lax.shift_right_logical(s16, 1)
            dlv[slot, pl.ds(k * 16, 16)] = (
                lax.shift_right_logical(dl16, 1)
                | ((dl16 & 1) << 14)
                | ((s16 & 1) << 22))

    def fire(bb, slot):
        @pl.when(slot == 0)
        def _():
            prep(bb, 0)
            pltpu.async_copy(shared.at[srcv.at[0]], rows.at[0], sem0)

        @pl.when(slot != 0)
        def _():
            prep(bb, 1)
            pltpu.async_copy(shared.at[srcv.at[1]], rows.at[1], sem1)

    def wait(slot):
        @pl.when(slot == 0)
        def _():
            pltpu.make_async_copy(shared.at[srcv.at[0]], rows.at[0],
                                  sem0).wait()

        @pl.when(slot != 0)
        def _():
            pltpu.make_async_copy(shared.at[srcv.at[1]], rows.at[1],
                                  sem1).wait()

    @pl.when(nb >= 1)
    def _():
        load_pk_chunk(0)
        fire(0, 0)

    def block_body(bb, _):
        slot = bb & 1

        @pl.when(bb + 1 < nb)
        def _():
            @pl.when(((bb + 1) & 15) == 0)
            def _():
                load_pk_chunk((bb + 1) >> 4)

            fire(bb + 1, slot ^ 1)

        wait(slot)

        def accum(rows_s, dlv_s):
            def qbody(q, _):
                dl16 = dlv_s[pl.ds(q * 16, 16)]
                val = dl16[0]
                for j in range(16):
                    qj = q * 16 + j
                    val_next = dl16[j + 1] if j < 15 else val
                    adl = val & 511
                    acl = lax.shift_right_logical(val, 8) & 64
                    cb = lax.shift_right_logical(val, 16) & 64
                    av = [acc[adl, pl.ds(acl, 16)],
                          acc[adl, pl.ds(acl + 16, 16)]]
                    rv = [rows_s[qj, pl.ds(cb, 16)],
                          rows_s[qj, pl.ds(cb + 16, 16)]]
                    for k in range(4):
                        if k + 2 < 4:
                            av.append(acc[adl, pl.ds(acl + (k + 2) * 16,
                                                     16)])
                            rv.append(rows_s[qj, pl.ds(cb + (k + 2) * 16,
                                                       16)])
                        m = jnp.maximum(
                            plsc.bitcast(av[k], jnp.bfloat16),
                            plsc.bitcast(rv[k], jnp.bfloat16))
                        acc[adl, pl.ds(acl + k * 16, 16)] = plsc.bitcast(
                            m, jnp.int32)
                    val = val_next
                return 0

            lax.fori_loop(0, GB // 16, qbody, 0)

        @pl.when(slot == 0)
        def _():
            accum(rows.at[0], dlv.at[0])

        @pl.when(slot != 0)
        def _():
            accum(rows.at[1], dlv.at[1])

        return 0

    lax.fori_loop(0, nb, block_body, 0)
    pltpu.sync_copy(acc.at[pl.ds(0, NPS // 2)],
                    out_ref.at[pl.ds(pl.multiple_of(w * (NPS // 2), 8),
                                     NPS // 2)])


_segmax = pl.kernel(
    _segmax_body,
    compiler_params=pltpu.CompilerParams(needs_layout_passes=False),
    out_type=jax.ShapeDtypeStruct((NPAD // 2, 2 * HW), jnp.int32),
    mesh=_mesh,
    scratch_types=[
        pltpu.VMEM((16 * GB,), jnp.int32),
        pltpu.VMEM((2, GB), jnp.int32),
        pltpu.VMEM((2, GB), jnp.int32),
        pltpu.VMEM((2, GB, 2 * HW), jnp.int32),
        pltpu.VMEM((ACC_ROWS, 2 * HW), jnp.int32),
        pltpu.VMEM((16,), jnp.int32),
        pltpu.VMEM_SHARED((NPAD // 2, 2 * HW), jnp.int32),
        pltpu.SemaphoreType.DMA,
        pltpu.SemaphoreType.DMA,
    ],
)


_RB = 2000


def _pool_body(x_ref, wp_ref, bp_ref, ws_ref, b_ref, pa_ref, pb_ref, s_ref):
    x = x_ref[...]
    p = jnp.maximum(
        jnp.dot(x, wp_ref[...], preferred_element_type=jnp.float32)
        + bp_ref[...], 0.0)

    def pack(lo_f, hi_f):
        lo = lax.bitcast_convert_type(lo_f.astype(jnp.bfloat16),
                                      jnp.uint16).astype(jnp.uint32)
        hi = lax.bitcast_convert_type(hi_f.astype(jnp.bfloat16),
                                      jnp.uint16).astype(jnp.uint32)
        return (lo | (hi << 16)).astype(jnp.int32)

    pa_ref[...] = pack(p[:, 0 * HW:1 * HW], p[:, 2 * HW:3 * HW])
    pb_ref[...] = pack(p[:, 1 * HW:2 * HW], p[:, 3 * HW:4 * HW])
    s_ref[...] = (jnp.dot(x, ws_ref[...], preferred_element_type=jnp.float32)
                  + b_ref[...])


def _pool(x, wp, bp, ws, b):
    return pl.pallas_call(
        _pool_body,
        grid=(N // _RB,),
        in_specs=[
            pl.BlockSpec((_RB, D), lambda i: (i, 0)),
            pl.BlockSpec((D, D), lambda i: (0, 0)),
            pl.BlockSpec((1, D), lambda i: (0, 0)),
            pl.BlockSpec((D, D), lambda i: (0, 0)),
            pl.BlockSpec((1, D), lambda i: (0, 0)),
        ],
        out_specs=[
            pl.BlockSpec((_RB, HW), lambda i: (i, 0)),
            pl.BlockSpec((_RB, HW), lambda i: (i, 0)),
            pl.BlockSpec((_RB, D), lambda i: (i, 0)),
        ],
        out_shape=[
            jax.ShapeDtypeStruct((N, HW), jnp.int32),
            jax.ShapeDtypeStruct((N, HW), jnp.int32),
            jax.ShapeDtypeStruct((N, D), jnp.float32),
        ],
    )(x, wp, bp.reshape(1, D), ws, b.reshape(1, D))


def _comb_body(s_ref, ga_ref, gb_ref, wn_ref, o_ref, *, act):
    def unpack(g_ref):
        g32 = g_ref[...].astype(jnp.uint32)
        lo = lax.bitcast_convert_type((g32 & 0xFFFF).astype(jnp.uint16),
                                      jnp.bfloat16)
        hi = lax.bitcast_convert_type((g32 >> 16).astype(jnp.uint16),
                                      jnp.bfloat16)
        return lo, hi

    alo, ahi = unpack(ga_ref)
    blo, bhi = unpack(gb_ref)
    g = jnp.concatenate([alo, blo, ahi, bhi], axis=1).astype(jnp.float32)
    o = s_ref[...] + jnp.dot(g, wn_ref[...],
                             preferred_element_type=jnp.float32)
    o_ref[...] = jnp.tanh(o) if act else o


def _comb(s, ga, gb, wn, act):
    return pl.pallas_call(
        functools.partial(_comb_body, act=act),
        grid=(N // _RB,),
        in_specs=[
            pl.BlockSpec((_RB, D), lambda i: (i, 0)),
            pl.BlockSpec((_RB, HW), lambda i: (i, 0)),
            pl.BlockSpec((_RB, HW), lambda i: (i, 0)),
            pl.BlockSpec((D, D), lambda i: (0, 0)),
        ],
        out_specs=pl.BlockSpec((_RB, D), lambda i: (i, 0)),
        out_shape=jax.ShapeDtypeStruct((N, D), jnp.float32),
    )(s, ga, gb, wn)


def kernel(x, edge_index,
           W_pool1, b_pool1, W_self1, W_neigh1, b1,
           W_pool2, b_pool2, W_self2, W_neigh2, b2,
           W_pool3, b_pool3, W_self3, W_neigh3, b3):
    bucket, counts = _bucket(edge_index[0], edge_index[1])

    def layer(h, wp, bp, ws, wn, b, act):
        pa, pb, s = _pool(h, wp, bp, ws, b)
        pa = jnp.pad(pa.reshape(N // 2, 2 * HW), ((0, (NPAD - N) // 2),
                                                  (0, 0)))
        pb = jnp.pad(pb.reshape(N // 2, 2 * HW), ((0, (NPAD - N) // 2),
                                                  (0, 0)))
        ga = _segmax(pa, bucket, counts)
        pb, ga = lax.optimization_barrier((pb, ga))
        gb = _segmax(pb, bucket, counts)
        ga = ga.reshape(NPAD, HW)[:N]
        gb = gb.reshape(NPAD, HW)[:N]
        return _comb(s, ga, gb, wn, act)

    h = layer(x, W_pool1, b_pool1, W_self1, W_neigh1, b1, True)
    h = layer(h, W_pool2, b_pool2, W_self2, W_neigh2, b2, True)
    h = layer(h, W_pool3, b_pool3, W_self3, W_neigh3, b3, False)
    return h
